# baseline (speedup 1.0000x reference)
import jax
import jax.numpy as jnp
from jax.experimental import pallas as pl

B, N = 64, 24
NF, EF = 64, 4
HID, MSG, PASSES = 64, 32, 3

Q = 4
G = B // Q
GB = 16

_DT = (((0,), (0,)), ((), ()))


def _dgT(a, b):
    return jax.lax.dot_general(a, b, _DT, preferred_element_type=jnp.float32)


def _dot(a, b):
    return jnp.dot(a, b, preferred_element_type=jnp.float32)


def _bd_mask(rows, cols, rblk, cblk):
    r = jax.lax.broadcasted_iota(jnp.int32, (rows, cols), 0)
    c = jax.lax.broadcasted_iota(jnp.int32, (rows, cols), 1)
    return (r // rblk == c // cblk).astype(jnp.float32)


def _mpnn_kernel(x_ref, et_ref, W_msg_ref, b_msg_ref, W_u_ref, W_m_ref,
                 b_u_ref, W_g_ref, W_o_ref, out_ref):
    f32 = jnp.float32
    NN = N * N
    QM, QH = Q * MSG, Q * HID
    QE = Q * EF
    S0t = _bd_mask(NN, N, N, 1)
    g_i = jax.lax.broadcasted_iota(jnp.int32, (N, NN), 0)
    r_i = jax.lax.broadcasted_iota(jnp.int32, (N, NN), 1)
    TileGt = (r_i % N == g_i).astype(f32)
    BlkSum = _bd_mask(Q, QE, 1, EF)
    BlkOnes64 = _bd_mask(QH, Q, HID, 1)
    A32t = _bd_mask(QM, QE, MSG, EF)
    OnesN1 = jnp.ones((N, 1), f32)
    Ones11 = jnp.ones((1, 1), f32)
    EyeN = (jax.lax.broadcasted_iota(jnp.int32, (N, N), 0) ==
            jax.lax.broadcasted_iota(jnp.int32, (N, N), 1)).astype(f32)

    W_h = W_msg_ref[0:HID, :]
    W_e = W_msg_ref[HID:HID + EF, :]
    b_msg = b_msg_ref[...]
    W_u = W_u_ref[...]
    W_m = W_m_ref[...]
    b_u = b_u_ref[...]
    W_gh = W_g_ref[0:HID, :]
    W_gx = W_g_ref[HID:2 * HID, :]
    W_oh = W_o_ref[0:HID, :]
    W_ox = W_o_ref[HID:2 * HID, :]

    WkH = jnp.tile(W_h, (Q, Q)) * _bd_mask(QH, QM, HID, MSG)
    WkE = jnp.tile(W_e, (Q, Q)) * _bd_mask(QE, QM, EF, MSG)
    WkU = jnp.tile(W_u, (Q, Q)) * _bd_mask(QH, QH, HID, HID)
    WkM = jnp.tile(W_m, (Q, Q)) * _bd_mask(QM, QH, MSG, HID)
    WkUM = jnp.concatenate([WkU, WkM], axis=0)
    m64 = _bd_mask(QH, QH, HID, HID)
    WkG = jnp.concatenate([jnp.tile(W_gh, (Q, Q)) * m64,
                           jnp.tile(W_gx, (Q, Q)) * m64], axis=0)
    WkO = jnp.concatenate([jnp.tile(W_oh, (Q, Q)) * m64,
                           jnp.tile(W_ox, (Q, Q)) * m64], axis=0)
    b_msg4t = _dgT(jnp.tile(b_msg, (1, Q)), Ones11)
    b_u4t = _dgT(jnp.tile(b_u, (1, Q)), Ones11)

    ets = [et_ref[gb] for gb in range(GB)]
    xts = [jnp.concatenate(
        [_dgT(x_ref[gb * Q + q], EyeN) for q in range(Q)], axis=0)
        for gb in range(GB)]
    E4s = [_dgT(WkE, et) + b_msg4t for et in ets]
    mask4s = [(_dot(A32t, et) != 0.0).astype(f32) for et in ets]
    gsums = [_dot(et, S0t) for et in ets]
    act4s = [_dot(BlkOnes64, _dot(BlkSum, gs)) != 0.0 for gs in gsums]

    hs = list(xts)
    for _ in range(PASSES):
        Hms = [_dgT(WkH, h) for h in hs]
        Ts = [jnp.tanh(E4s[i] + _dot(Hms[i], TileGt))
              for i in range(GB)]
        msgs = [_dot(Ts[i] * mask4s[i], S0t) for i in range(GB)]
        hs = [jnp.where(
            act4s[i],
            jnp.tanh(_dgT(WkUM,
                          jnp.concatenate([hs[i], msgs[i]], axis=0))
                     + b_u4t),
            hs[i]) for i in range(GB)]

    for gb in range(GB):
        hx = jnp.concatenate([hs[gb], xts[gb]], axis=0)
        gate = jax.nn.sigmoid(_dgT(WkG, hx))
        o = _dgT(WkO, hx)
        gated = gate * o * act4s[gb].astype(f32)
        out_ref[gb] = _dot(gated, OnesN1)


def kernel(nodes, edges, W_msg, b_msg, W_u, W_m, b_u, W_g, W_o):
    f32 = jnp.float32
    et = edges.reshape(G, Q, N * N, EF).transpose(0, 1, 3, 2) \
              .reshape(G, Q * EF, N * N)
    b_msg2 = b_msg.reshape(1, MSG)
    b_u2 = b_u.reshape(1, HID)

    full = lambda shape: pl.BlockSpec(shape, lambda i: (0,) * len(shape))
    out = pl.pallas_call(
        _mpnn_kernel,
        grid=(G // GB,),
        in_specs=[
            pl.BlockSpec((GB * Q, N, NF), lambda i: (i, 0, 0)),
            pl.BlockSpec((GB, Q * EF, N * N), lambda i: (i, 0, 0)),
            full((HID + EF, MSG)),
            full((1, MSG)),
            full((HID, HID)),
            full((MSG, HID)),
            full((1, HID)),
            full((2 * HID, HID)),
            full((2 * HID, HID)),
        ],
        out_specs=pl.BlockSpec((GB, Q * HID, 1), lambda i: (i, 0, 0)),
        out_shape=jax.ShapeDtypeStruct((G, Q * HID, 1), f32),
    )(nodes, et, W_msg, b_msg2, W_u, W_m, b_u2, W_g, W_o)
    return out.reshape(B, HID)


# speedup vs baseline: 1.0096x; 1.0096x over previous
import jax
import jax.numpy as jnp
from jax.experimental import pallas as pl
from jax.experimental.pallas import tpu as pltpu

B, N = 64, 24
NF, EF = 64, 4
HID, MSG, PASSES = 64, 32, 3

Q = 4
G = B // Q
GB = 8

_DT = (((0,), (0,)), ((), ()))


def _dgT(a, b):
    return jax.lax.dot_general(a, b, _DT, preferred_element_type=jnp.float32)


def _dot(a, b):
    return jnp.dot(a, b, preferred_element_type=jnp.float32)


def _bd_mask(rows, cols, rblk, cblk):
    r = jax.lax.broadcasted_iota(jnp.int32, (rows, cols), 0)
    c = jax.lax.broadcasted_iota(jnp.int32, (rows, cols), 1)
    return (r // rblk == c // cblk).astype(jnp.float32)


def _mpnn_kernel(x_ref, et_ref, W_msg_ref, b_msg_ref, W_u_ref, W_m_ref,
                 b_u_ref, W_g_ref, W_o_ref, out_ref):
    f32 = jnp.float32
    NN = N * N
    QM, QH = Q * MSG, Q * HID
    QE = Q * EF
    S0t = _bd_mask(NN, N, N, 1)
    g_i = jax.lax.broadcasted_iota(jnp.int32, (N, NN), 0)
    r_i = jax.lax.broadcasted_iota(jnp.int32, (N, NN), 1)
    TileGt = (r_i % N == g_i).astype(f32)
    BlkSum = _bd_mask(Q, QE, 1, EF)
    BlkOnes64 = _bd_mask(QH, Q, HID, 1)
    A32t = _bd_mask(QM, QE, MSG, EF)
    OnesN1 = jnp.ones((N, 1), f32)
    Ones11 = jnp.ones((1, 1), f32)
    EyeN = (jax.lax.broadcasted_iota(jnp.int32, (N, N), 0) ==
            jax.lax.broadcasted_iota(jnp.int32, (N, N), 1)).astype(f32)

    W_h = W_msg_ref[0:HID, :]
    W_e = W_msg_ref[HID:HID + EF, :]
    b_msg = b_msg_ref[...]
    W_u = W_u_ref[...]
    W_m = W_m_ref[...]
    b_u = b_u_ref[...]
    W_gh = W_g_ref[0:HID, :]
    W_gx = W_g_ref[HID:2 * HID, :]
    W_oh = W_o_ref[0:HID, :]
    W_ox = W_o_ref[HID:2 * HID, :]

    WkH = jnp.tile(W_h, (Q, Q)) * _bd_mask(QH, QM, HID, MSG)
    WkE = jnp.tile(W_e, (Q, Q)) * _bd_mask(QE, QM, EF, MSG)
    WkU = jnp.tile(W_u, (Q, Q)) * _bd_mask(QH, QH, HID, HID)
    WkM = jnp.tile(W_m, (Q, Q)) * _bd_mask(QM, QH, MSG, HID)
    WkUM = jnp.concatenate([WkU, WkM], axis=0)
    m64 = _bd_mask(QH, QH, HID, HID)
    WkG = jnp.concatenate([jnp.tile(W_gh, (Q, Q)) * m64,
                           jnp.tile(W_gx, (Q, Q)) * m64], axis=0)
    WkO = jnp.concatenate([jnp.tile(W_oh, (Q, Q)) * m64,
                           jnp.tile(W_ox, (Q, Q)) * m64], axis=0)
    b_msg4t = _dgT(jnp.tile(b_msg, (1, Q)), Ones11)
    b_u4t = _dgT(jnp.tile(b_u, (1, Q)), Ones11)

    ets = [et_ref[gb] for gb in range(GB)]
    xts = [jnp.concatenate(
        [_dgT(x_ref[gb * Q + q], EyeN) for q in range(Q)], axis=0)
        for gb in range(GB)]
    E4s = [_dgT(WkE, et) + b_msg4t for et in ets]
    mask4s = [(_dot(A32t, et) != 0.0).astype(f32) for et in ets]
    gsums = [_dot(et, S0t) for et in ets]
    act4s = [_dot(BlkOnes64, _dot(BlkSum, gs)) != 0.0 for gs in gsums]

    hs = list(xts)
    for _ in range(PASSES):
        Hms = [_dgT(WkH, h) for h in hs]
        Ts = [jnp.tanh(E4s[i] + _dot(Hms[i], TileGt))
              for i in range(GB)]
        msgs = [_dot(Ts[i] * mask4s[i], S0t) for i in range(GB)]
        hs = [jnp.where(
            act4s[i],
            jnp.tanh(_dgT(WkUM,
                          jnp.concatenate([hs[i], msgs[i]], axis=0))
                     + b_u4t),
            hs[i]) for i in range(GB)]

    for gb in range(GB):
        hx = jnp.concatenate([hs[gb], xts[gb]], axis=0)
        gate = jax.nn.sigmoid(_dgT(WkG, hx))
        o = _dgT(WkO, hx)
        gated = gate * o * act4s[gb].astype(f32)
        out_ref[gb] = _dot(gated, OnesN1)


def kernel(nodes, edges, W_msg, b_msg, W_u, W_m, b_u, W_g, W_o):
    f32 = jnp.float32
    et = edges.reshape(G, Q, N * N, EF).transpose(0, 1, 3, 2) \
              .reshape(G, Q * EF, N * N)
    b_msg2 = b_msg.reshape(1, MSG)
    b_u2 = b_u.reshape(1, HID)

    full = lambda shape: pl.BlockSpec(shape, lambda i: (0,) * len(shape))
    out = pl.pallas_call(
        _mpnn_kernel,
        grid=(G // GB,),
        in_specs=[
            pl.BlockSpec((GB * Q, N, NF), lambda i: (i, 0, 0)),
            pl.BlockSpec((GB, Q * EF, N * N), lambda i: (i, 0, 0)),
            full((HID + EF, MSG)),
            full((1, MSG)),
            full((HID, HID)),
            full((MSG, HID)),
            full((1, HID)),
            full((2 * HID, HID)),
            full((2 * HID, HID)),
        ],
        out_specs=pl.BlockSpec((GB, Q * HID, 1), lambda i: (i, 0, 0)),
        out_shape=jax.ShapeDtypeStruct((G, Q * HID, 1), f32),
        compiler_params=pltpu.CompilerParams(
            dimension_semantics=("parallel",)),
    )(nodes, et, W_msg, b_msg2, W_u, W_m, b_u2, W_g, W_o)
    return out.reshape(B, HID)


# speedup vs baseline: 1.1154x; 1.1048x over previous
import jax
import jax.numpy as jnp
from jax.experimental import pallas as pl
from jax.experimental.pallas import tpu as pltpu

B, N = 64, 24
NF, EF = 64, 4
HID, MSG, PASSES = 64, 32, 3

Q = 4
G = B // Q
GB = 8

_DT = (((0,), (0,)), ((), ()))


def _dgT(a, b):
    return jax.lax.dot_general(a, b, _DT, preferred_element_type=jnp.float32)


def _dot(a, b):
    return jnp.dot(a, b, preferred_element_type=jnp.float32)


def _bd_mask(rows, cols, rblk, cblk):
    r = jax.lax.broadcasted_iota(jnp.int32, (rows, cols), 0)
    c = jax.lax.broadcasted_iota(jnp.int32, (rows, cols), 1)
    return (r // rblk == c // cblk).astype(jnp.float32)


def _mpnn_kernel(x_ref, et_ref, W_msg_ref, b_msg_ref, W_u_ref, W_m_ref,
                 b_u_ref, W_g_ref, W_o_ref, out_ref):
    f32 = jnp.float32
    NN = N * N
    QM, QH = Q * MSG, Q * HID
    QE = Q * EF
    S0t = _bd_mask(NN, N, N, 1)
    g_i = jax.lax.broadcasted_iota(jnp.int32, (N, NN), 0)
    r_i = jax.lax.broadcasted_iota(jnp.int32, (N, NN), 1)
    TileGt = (r_i % N == g_i).astype(f32)
    BlkSum = _bd_mask(Q, QE, 1, EF)
    BlkOnes64 = _bd_mask(QH, Q, HID, 1)
    A32t = _bd_mask(QM, QE, MSG, EF)
    OnesN1 = jnp.ones((N, 1), f32)
    Ones11 = jnp.ones((1, 1), f32)
    EyeN = (jax.lax.broadcasted_iota(jnp.int32, (N, N), 0) ==
            jax.lax.broadcasted_iota(jnp.int32, (N, N), 1)).astype(f32)

    W_h = W_msg_ref[0:HID, :]
    W_e = W_msg_ref[HID:HID + EF, :]
    b_msg = b_msg_ref[...]
    W_u = W_u_ref[...]
    W_m = W_m_ref[...]
    b_u = b_u_ref[...]
    W_gh = W_g_ref[0:HID, :]
    W_gx = W_g_ref[HID:2 * HID, :]
    W_oh = W_o_ref[0:HID, :]
    W_ox = W_o_ref[HID:2 * HID, :]

    WkH = jnp.tile(W_h, (Q, Q)) * _bd_mask(QH, QM, HID, MSG)
    WkE = jnp.tile(W_e, (Q, Q)) * _bd_mask(QE, QM, EF, MSG)
    W_um = jnp.concatenate([W_u, W_m], axis=0)
    b_msg4t = _dgT(jnp.tile(b_msg, (1, Q)), Ones11)
    b_u4t = _dgT(jnp.tile(b_u, (1, Q)), Ones11)

    ets = [et_ref[gb] for gb in range(GB)]
    xts = [jnp.concatenate(
        [_dgT(x_ref[gb * Q + q], EyeN) for q in range(Q)], axis=0)
        for gb in range(GB)]
    E4s = [_dgT(WkE, et) + b_msg4t for et in ets]
    mask4s = [(_dot(A32t, et) != 0.0).astype(f32) for et in ets]
    gsums = [_dot(et, S0t) for et in ets]
    act4s = [_dot(BlkOnes64, _dot(BlkSum, gs)) != 0.0 for gs in gsums]

    hs = list(xts)
    for _ in range(PASSES):
        Hms = [_dgT(WkH, h) for h in hs]
        Ts = [jnp.tanh(E4s[i] + _dot(Hms[i], TileGt))
              for i in range(GB)]
        msgs = [_dot(Ts[i] * mask4s[i], S0t) for i in range(GB)]
        pre = [jnp.concatenate(
            [_dgT(W_um, jnp.concatenate(
                [hs[i][q * HID:(q + 1) * HID, :],
                 msgs[i][q * MSG:(q + 1) * MSG, :]], axis=0))
             for q in range(Q)], axis=0) for i in range(GB)]
        hs = [jnp.where(act4s[i], jnp.tanh(pre[i] + b_u4t), hs[i])
              for i in range(GB)]

    W_go = jnp.concatenate([W_g_ref[...], W_o_ref[...]], axis=1)
    for gb in range(GB):
        gos = []
        for q in range(Q):
            hx_q = jnp.concatenate(
                [hs[gb][q * HID:(q + 1) * HID, :],
                 xts[gb][q * HID:(q + 1) * HID, :]], axis=0)
            gos.append(_dgT(W_go, hx_q))
        gate = jnp.concatenate(
            [jax.nn.sigmoid(gos[q][0:HID, :]) for q in range(Q)], axis=0)
        o = jnp.concatenate([gos[q][HID:2 * HID, :] for q in range(Q)],
                            axis=0)
        gated = gate * o * act4s[gb].astype(f32)
        out_ref[gb] = _dot(gated, OnesN1)


def kernel(nodes, edges, W_msg, b_msg, W_u, W_m, b_u, W_g, W_o):
    f32 = jnp.float32
    et = edges.reshape(G, Q, N * N, EF).transpose(0, 1, 3, 2) \
              .reshape(G, Q * EF, N * N)
    b_msg2 = b_msg.reshape(1, MSG)
    b_u2 = b_u.reshape(1, HID)

    full = lambda shape: pl.BlockSpec(shape, lambda i: (0,) * len(shape))
    out = pl.pallas_call(
        _mpnn_kernel,
        grid=(G // GB,),
        in_specs=[
            pl.BlockSpec((GB * Q, N, NF), lambda i: (i, 0, 0)),
            pl.BlockSpec((GB, Q * EF, N * N), lambda i: (i, 0, 0)),
            full((HID + EF, MSG)),
            full((1, MSG)),
            full((HID, HID)),
            full((MSG, HID)),
            full((1, HID)),
            full((2 * HID, HID)),
            full((2 * HID, HID)),
        ],
        out_specs=pl.BlockSpec((GB, Q * HID, 1), lambda i: (i, 0, 0)),
        out_shape=jax.ShapeDtypeStruct((G, Q * HID, 1), f32),
        compiler_params=pltpu.CompilerParams(
            dimension_semantics=("parallel",)),
    )(nodes, et, W_msg, b_msg2, W_u, W_m, b_u2, W_g, W_o)
    return out.reshape(B, HID)


# speedup vs baseline: 1.1170x; 1.0014x over previous
import jax
import jax.numpy as jnp
from jax.experimental import pallas as pl
from jax.experimental.pallas import tpu as pltpu

B, N = 64, 24
NF, EF = 64, 4
HID, MSG, PASSES = 64, 32, 3

Q = 4
G = B // Q
GB = 8

_DT = (((0,), (0,)), ((), ()))


def _dgT(a, b):
    return jax.lax.dot_general(a, b, _DT, preferred_element_type=jnp.float32)


def _dot(a, b):
    return jnp.dot(a, b, preferred_element_type=jnp.float32)


def _bd_mask(rows, cols, rblk, cblk):
    r = jax.lax.broadcasted_iota(jnp.int32, (rows, cols), 0)
    c = jax.lax.broadcasted_iota(jnp.int32, (rows, cols), 1)
    return (r // rblk == c // cblk).astype(jnp.float32)


def _mpnn_kernel(x_ref, et_ref, W_msg_ref, b_msg_ref, W_u_ref, W_m_ref,
                 b_u_ref, W_g_ref, W_o_ref, out_ref):
    f32 = jnp.float32
    NN = N * N
    QM, QH = Q * MSG, Q * HID
    QE = Q * EF
    S0t = _bd_mask(NN, N, N, 1)
    g_i = jax.lax.broadcasted_iota(jnp.int32, (N, NN), 0)
    r_i = jax.lax.broadcasted_iota(jnp.int32, (N, NN), 1)
    TileGt = (r_i % N == g_i).astype(f32)
    BlkSum = _bd_mask(Q, QE, 1, EF)
    BlkOnes64 = _bd_mask(QH, Q, HID, 1)
    A32t = _bd_mask(QM, QE, MSG, EF)
    OnesN1 = jnp.ones((N, 1), f32)
    Ones11 = jnp.ones((1, 1), f32)
    EyeN = (jax.lax.broadcasted_iota(jnp.int32, (N, N), 0) ==
            jax.lax.broadcasted_iota(jnp.int32, (N, N), 1)).astype(f32)

    W_h = W_msg_ref[0:HID, :]
    W_e = W_msg_ref[HID:HID + EF, :]
    b_msg = b_msg_ref[...]
    W_u = W_u_ref[...]
    W_m = W_m_ref[...]
    b_u = b_u_ref[...]

    WkH = jnp.tile(W_h, (Q, Q)) * _bd_mask(QH, QM, HID, MSG)
    WkE = jnp.tile(W_e, (Q, Q)) * _bd_mask(QE, QM, EF, MSG)
    W_um = jnp.concatenate([W_u, W_m], axis=0)
    b_msg4t = _dgT(jnp.tile(b_msg, (1, Q)), Ones11)
    b_u4t = _dgT(jnp.tile(b_u, (1, Q)), Ones11)

    ets = [et_ref[gb] for gb in range(GB)]
    xts = [jnp.concatenate(
        [_dgT(x_ref[gb * Q + q], EyeN) for q in range(Q)], axis=0)
        for gb in range(GB)]
    E4s = [_dgT(WkE, et) + b_msg4t for et in ets]
    mask4s = [(_dot(A32t, et) != 0.0).astype(f32) for et in ets]
    gsums = [_dot(et, S0t) for et in ets]
    act4s = [_dot(BlkOnes64, _dot(BlkSum, gs)) != 0.0 for gs in gsums]

    hs = list(xts)
    for _ in range(PASSES):
        Hms = [_dgT(WkH, h) for h in hs]
        Ts = [jnp.tanh(E4s[i] + _dot(Hms[i], TileGt))
              for i in range(GB)]
        msgs = [_dot(Ts[i] * mask4s[i], S0t) for i in range(GB)]
        pre = [jnp.concatenate(
            [_dgT(W_um, jnp.concatenate(
                [hs[i][q * HID:(q + 1) * HID, :],
                 msgs[i][q * MSG:(q + 1) * MSG, :]], axis=0))
             for q in range(Q)], axis=0) for i in range(GB)]
        hs = [jnp.where(act4s[i], jnp.tanh(pre[i] + b_u4t), hs[i])
              for i in range(GB)]

    W_go = jnp.concatenate([W_g_ref[...], W_o_ref[...]], axis=1)
    for gb in range(GB):
        gos = []
        for q in range(Q):
            hx_q = jnp.concatenate(
                [hs[gb][q * HID:(q + 1) * HID, :],
                 xts[gb][q * HID:(q + 1) * HID, :]], axis=0)
            gos.append(_dgT(W_go, hx_q))
        gate = jnp.concatenate(
            [jax.nn.sigmoid(gos[q][0:HID, :]) for q in range(Q)], axis=0)
        o = jnp.concatenate([gos[q][HID:2 * HID, :] for q in range(Q)],
                            axis=0)
        gated = gate * o * act4s[gb].astype(f32)
        out_ref[gb] = _dot(gated, OnesN1)


def kernel(nodes, edges, W_msg, b_msg, W_u, W_m, b_u, W_g, W_o):
    f32 = jnp.float32
    et = edges.reshape(G, Q, N * N, EF).transpose(0, 1, 3, 2) \
              .reshape(G, Q * EF, N * N)
    b_msg2 = b_msg.reshape(1, MSG)
    b_u2 = b_u.reshape(1, HID)

    full = lambda shape: pl.BlockSpec(shape, lambda i: (0,) * len(shape))
    out = pl.pallas_call(
        _mpnn_kernel,
        grid=(G // GB,),
        in_specs=[
            pl.BlockSpec((GB * Q, N, NF), lambda i: (i, 0, 0)),
            pl.BlockSpec((GB, Q * EF, N * N), lambda i: (i, 0, 0)),
            full((HID + EF, MSG)),
            full((1, MSG)),
            full((HID, HID)),
            full((MSG, HID)),
            full((1, HID)),
            full((2 * HID, HID)),
            full((2 * HID, HID)),
        ],
        out_specs=pl.BlockSpec((GB, Q * HID, 1), lambda i: (i, 0, 0)),
        out_shape=jax.ShapeDtypeStruct((G, Q * HID, 1), f32),
        compiler_params=pltpu.CompilerParams(
            dimension_semantics=("parallel",)),
    )(nodes, et, W_msg, b_msg2, W_u, W_m, b_u2, W_g, W_o)
    return out.reshape(B, HID)
